# initial kernel scaffold (unmeasured)
import jax
import jax.numpy as jnp
from jax import lax
from jax.experimental import pallas as pl
from jax.experimental.pallas import tpu as pltpu

N_DEV = 4
T = 512
V_LOC = 8192


def kernel(x, W, labels):
    def body(x_ref, w_ref, lab_ref, out_ref, comm_ref, send_sems, recv_sems):
        my_pos = lax.axis_index("i")

        barrier_sem = pltpu.get_barrier_semaphore()
        for d in (1, 2, 3):
            pl.semaphore_signal(
                barrier_sem, inc=1,
                device_id=((my_pos + d) % N_DEV,),
                device_id_type=pl.DeviceIdType.MESH,
            )

        logits = jnp.dot(
            x_ref[:, :], w_ref[:, :], preferred_element_type=jnp.float32
        )
        m = jnp.max(logits, axis=1)
        s = jnp.sum(jnp.exp(logits - m[:, None]), axis=1)
        col = lax.broadcasted_iota(jnp.int32, (T, V_LOC), 1) + my_pos * V_LOC
        hit = col == lab_ref[:, :]
        lab_part = jnp.sum(jnp.where(hit, logits, 0.0), axis=1)

        comm_ref[my_pos, 0, :] = m
        comm_ref[my_pos, 1, :] = s
        comm_ref[my_pos, 2, :] = lab_part

        pl.semaphore_wait(barrier_sem, 3)

        rdmas = []
        for d in (1, 2, 3):
            rdma = pltpu.make_async_remote_copy(
                src_ref=comm_ref.at[my_pos],
                dst_ref=comm_ref.at[my_pos],
                send_sem=send_sems.at[d],
                recv_sem=recv_sems.at[d],
                device_id=((my_pos + d) % N_DEV,),
                device_id_type=pl.DeviceIdType.MESH,
            )
            rdma.start()
            rdmas.append(rdma)
        for rdma in rdmas:
            rdma.wait()

        stats = comm_ref[:, :, :]
        m_all = stats[:, 0, :]
        s_all = stats[:, 1, :]
        l_all = stats[:, 2, :]
        m_g = jnp.max(m_all, axis=0)
        s_g = jnp.sum(s_all * jnp.exp(m_all - m_g[None, :]), axis=0)
        lab_logit = jnp.sum(l_all, axis=0)
        out_ref[0, :] = m_g + jnp.log(s_g) - lab_logit

    out = pl.pallas_call(
        body,
        out_shape=jax.ShapeDtypeStruct((1, T), jnp.float32),
        in_specs=[
            pl.BlockSpec(memory_space=pltpu.VMEM),
            pl.BlockSpec(memory_space=pltpu.VMEM),
            pl.BlockSpec(memory_space=pltpu.VMEM),
        ],
        out_specs=pl.BlockSpec(memory_space=pltpu.VMEM),
        scratch_shapes=[
            pltpu.VMEM((N_DEV, 3, T), jnp.float32),
            pltpu.SemaphoreType.DMA((N_DEV,)),
            pltpu.SemaphoreType.DMA((N_DEV,)),
        ],
        compiler_params=pltpu.CompilerParams(collective_id=0),
    )(x, W, labels.reshape(T, 1))
    return out.reshape(T)


# baseline (device time: 32375 ns/iter reference)
import jax
import jax.numpy as jnp
from jax import lax
from jax.experimental import pallas as pl
from jax.experimental.pallas import tpu as pltpu

N_DEV = 4
T = 512
V_LOC = 8192


def kernel(x, W, labels):
    def body(x_ref, w_ref, lab_ref, out_ref, comm_ref, send_sems, recv_sems):
        my_pos = lax.axis_index("i")

        barrier_sem = pltpu.get_barrier_semaphore()
        for d in (1, 2, 3):
            pl.semaphore_signal(
                barrier_sem, inc=1,
                device_id=((my_pos + d) % N_DEV,),
                device_id_type=pl.DeviceIdType.MESH,
            )

        logits = jnp.dot(
            x_ref[:, :], w_ref[:, :], preferred_element_type=jnp.float32
        )
        m = jnp.max(logits, axis=1)
        s = jnp.sum(jnp.exp(logits - m[:, None]), axis=1)
        col = lax.broadcasted_iota(jnp.int32, (T, V_LOC), 1) + my_pos * V_LOC
        hit = col == lab_ref[:, :]
        lab_part = jnp.sum(jnp.where(hit, logits, 0.0), axis=1)

        comm_ref[my_pos, 0, :] = m
        comm_ref[my_pos, 1, :] = s
        comm_ref[my_pos, 2, :] = lab_part

        pl.semaphore_wait(barrier_sem, 3)

        rdmas = []
        for d in (1, 2, 3):
            rdma = pltpu.make_async_remote_copy(
                src_ref=comm_ref.at[my_pos],
                dst_ref=comm_ref.at[my_pos],
                send_sem=send_sems.at[d],
                recv_sem=recv_sems.at[d],
                device_id=((my_pos + d) % N_DEV,),
                device_id_type=pl.DeviceIdType.MESH,
            )
            rdma.start()
            rdmas.append(rdma)
        for rdma in rdmas:
            rdma.wait()

        stats = comm_ref[:, :, :]
        m_all = stats[:, 0, :]
        s_all = stats[:, 1, :]
        l_all = stats[:, 2, :]
        m_g = jnp.max(m_all, axis=0)
        s_g = jnp.sum(s_all * jnp.exp(m_all - m_g[None, :]), axis=0)
        lab_logit = jnp.sum(l_all, axis=0)
        out_ref[0, :] = m_g + jnp.log(s_g) - lab_logit

    out = pl.pallas_call(
        body,
        out_shape=jax.ShapeDtypeStruct((1, T), jnp.float32),
        in_specs=[
            pl.BlockSpec(memory_space=pltpu.VMEM),
            pl.BlockSpec(memory_space=pltpu.VMEM),
            pl.BlockSpec(memory_space=pltpu.VMEM),
        ],
        out_specs=pl.BlockSpec(memory_space=pltpu.VMEM),
        scratch_shapes=[
            pltpu.VMEM((N_DEV, 3, T), jnp.float32),
            pltpu.SemaphoreType.DMA((N_DEV,)),
            pltpu.SemaphoreType.DMA((N_DEV,)),
        ],
        compiler_params=pltpu.CompilerParams(
            collective_id=0, vmem_limit_bytes=120 * 1024 * 1024
        ),
    )(x, W, labels.reshape(T, 1))
    return out.reshape(T)


# device time: 32200 ns/iter; 1.0054x vs baseline; 1.0054x over previous
import jax
import jax.numpy as jnp
from jax import lax
from jax.experimental import pallas as pl
from jax.experimental.pallas import tpu as pltpu

N_DEV = 4
T = 512
V_LOC = 8192
V_CHUNK = 1024
N_CHUNKS = V_LOC // V_CHUNK


def kernel(x, W, labels):
    def body(x_ref, w_ref, lab_ref, out_ref, comm_ref, send_sems, recv_sems):
        j = pl.program_id(0)
        my_pos = lax.axis_index("i")
        barrier_sem = pltpu.get_barrier_semaphore()

        @pl.when(j == 0)
        def _():
            for d in (1, 2, 3):
                pl.semaphore_signal(
                    barrier_sem, inc=1,
                    device_id=((my_pos + d) % N_DEV,),
                    device_id_type=pl.DeviceIdType.MESH,
                )

        logits = jnp.dot(
            x_ref[:, :], w_ref[:, :], preferred_element_type=jnp.float32
        )
        m_j = jnp.max(logits, axis=1)
        s_j = jnp.sum(jnp.exp(logits - m_j[:, None]), axis=1)
        col = (
            lax.broadcasted_iota(jnp.int32, (T, V_CHUNK), 1)
            + my_pos * V_LOC + j * V_CHUNK
        )
        lab_j = jnp.sum(
            jnp.where(col == lab_ref[:, :], logits, 0.0), axis=1
        )

        @pl.when(j == 0)
        def _():
            comm_ref[my_pos, 0, :] = m_j
            comm_ref[my_pos, 1, :] = s_j
            comm_ref[my_pos, 2, :] = lab_j

        @pl.when(j > 0)
        def _():
            m_old = comm_ref[my_pos, 0, :]
            s_old = comm_ref[my_pos, 1, :]
            m_new = jnp.maximum(m_old, m_j)
            comm_ref[my_pos, 0, :] = m_new
            comm_ref[my_pos, 1, :] = (
                s_old * jnp.exp(m_old - m_new) + s_j * jnp.exp(m_j - m_new)
            )
            comm_ref[my_pos, 2, :] = comm_ref[my_pos, 2, :] + lab_j

        @pl.when(j == N_CHUNKS - 1)
        def _():
            pl.semaphore_wait(barrier_sem, 3)

            rdmas = []
            for d in (1, 2, 3):
                rdma = pltpu.make_async_remote_copy(
                    src_ref=comm_ref.at[my_pos],
                    dst_ref=comm_ref.at[my_pos],
                    send_sem=send_sems.at[d],
                    recv_sem=recv_sems.at[d],
                    device_id=((my_pos + d) % N_DEV,),
                    device_id_type=pl.DeviceIdType.MESH,
                )
                rdma.start()
                rdmas.append(rdma)
            for rdma in rdmas:
                rdma.wait()

            stats = comm_ref[:, :, :]
            m_all = stats[:, 0, :]
            s_all = stats[:, 1, :]
            l_all = stats[:, 2, :]
            m_g = jnp.max(m_all, axis=0)
            s_g = jnp.sum(s_all * jnp.exp(m_all - m_g[None, :]), axis=0)
            lab_logit = jnp.sum(l_all, axis=0)
            out_ref[0, :] = m_g + jnp.log(s_g) - lab_logit

    out = pl.pallas_call(
        body,
        grid=(N_CHUNKS,),
        out_shape=jax.ShapeDtypeStruct((1, T), jnp.float32),
        in_specs=[
            pl.BlockSpec((T, 1024), lambda j: (0, 0)),
            pl.BlockSpec((1024, V_CHUNK), lambda j: (0, j)),
            pl.BlockSpec((T, 1), lambda j: (0, 0)),
        ],
        out_specs=pl.BlockSpec((1, T), lambda j: (0, 0)),
        scratch_shapes=[
            pltpu.VMEM((N_DEV, 3, T), jnp.float32),
            pltpu.SemaphoreType.DMA((N_DEV,)),
            pltpu.SemaphoreType.DMA((N_DEV,)),
        ],
        compiler_params=pltpu.CompilerParams(
            collective_id=0, vmem_limit_bytes=100 * 1024 * 1024
        ),
    )(x, W, labels.reshape(T, 1))
    return out.reshape(T)


# device time: 27578 ns/iter; 1.1739x vs baseline; 1.1676x over previous
import jax
import jax.numpy as jnp
from jax import lax
from jax.experimental import pallas as pl
from jax.experimental.pallas import tpu as pltpu

N_DEV = 4
T = 512
V_LOC = 8192
V_CHUNK = 1024
N_CHUNKS = V_LOC // V_CHUNK


def kernel(x, W, labels):
    def body(x_ref, w_hbm, lab_ref, out_ref,
             w_vmem, dma_sems, comm_ref, send_sems, recv_sems):
        my_pos = lax.axis_index("i")

        barrier_sem = pltpu.get_barrier_semaphore()
        for d in (1, 2, 3):
            pl.semaphore_signal(
                barrier_sem, inc=1,
                device_id=((my_pos + d) % N_DEV,),
                device_id_type=pl.DeviceIdType.MESH,
            )

        copies = []
        for c in range(N_CHUNKS):
            cp = pltpu.make_async_copy(
                w_hbm.at[:, pl.ds(c * V_CHUNK, V_CHUNK)],
                w_vmem.at[:, pl.ds(c * V_CHUNK, V_CHUNK)],
                dma_sems.at[c],
            )
            cp.start()
            copies.append(cp)

        x16 = x_ref[:, :].astype(jnp.bfloat16)
        lab = lab_ref[:, :]
        m_run = s_run = lab_run = None
        for c in range(N_CHUNKS):
            copies[c].wait()
            logits = jnp.dot(
                x16,
                w_vmem[:, c * V_CHUNK:(c + 1) * V_CHUNK].astype(jnp.bfloat16),
                preferred_element_type=jnp.float32,
            )
            m_c = jnp.max(logits, axis=1)
            s_c = jnp.sum(jnp.exp(logits - m_c[:, None]), axis=1)
            col = (
                lax.broadcasted_iota(jnp.int32, (T, V_CHUNK), 1)
                + my_pos * V_LOC + c * V_CHUNK
            )
            lab_c = jnp.sum(jnp.where(col == lab, logits, 0.0), axis=1)
            if c == 0:
                m_run, s_run, lab_run = m_c, s_c, lab_c
            else:
                m_new = jnp.maximum(m_run, m_c)
                s_run = (
                    s_run * jnp.exp(m_run - m_new)
                    + s_c * jnp.exp(m_c - m_new)
                )
                m_run = m_new
                lab_run = lab_run + lab_c

        comm_ref[my_pos, 0, :] = m_run
        comm_ref[my_pos, 1, :] = s_run
        comm_ref[my_pos, 2, :] = lab_run

        pl.semaphore_wait(barrier_sem, 3)

        rdmas = []
        for d in (1, 2, 3):
            rdma = pltpu.make_async_remote_copy(
                src_ref=comm_ref.at[my_pos],
                dst_ref=comm_ref.at[my_pos],
                send_sem=send_sems.at[d],
                recv_sem=recv_sems.at[d],
                device_id=((my_pos + d) % N_DEV,),
                device_id_type=pl.DeviceIdType.MESH,
            )
            rdma.start()
            rdmas.append(rdma)
        for rdma in rdmas:
            rdma.wait()

        stats = comm_ref[:, :, :]
        m_all = stats[:, 0, :]
        s_all = stats[:, 1, :]
        l_all = stats[:, 2, :]
        m_g = jnp.max(m_all, axis=0)
        s_g = jnp.sum(s_all * jnp.exp(m_all - m_g[None, :]), axis=0)
        lab_logit = jnp.sum(l_all, axis=0)
        out_ref[0, :] = m_g + jnp.log(s_g) - lab_logit

    out = pl.pallas_call(
        body,
        out_shape=jax.ShapeDtypeStruct((1, T), jnp.float32),
        in_specs=[
            pl.BlockSpec(memory_space=pltpu.VMEM),
            pl.BlockSpec(memory_space=pl.ANY),
            pl.BlockSpec(memory_space=pltpu.VMEM),
        ],
        out_specs=pl.BlockSpec(memory_space=pltpu.VMEM),
        scratch_shapes=[
            pltpu.VMEM((1024, V_LOC), jnp.float32),
            pltpu.SemaphoreType.DMA((N_CHUNKS,)),
            pltpu.VMEM((N_DEV, 3, T), jnp.float32),
            pltpu.SemaphoreType.DMA((N_DEV,)),
            pltpu.SemaphoreType.DMA((N_DEV,)),
        ],
        compiler_params=pltpu.CompilerParams(
            collective_id=0, vmem_limit_bytes=100 * 1024 * 1024
        ),
    )(x, W, labels.reshape(T, 1))
    return out.reshape(T)
